# Initial kernel scaffold; baseline (speedup 1.0000x reference)
import jax
import jax.numpy as jnp
from jax.experimental import pallas as pl


def kernel(token_types, bb_indices, rxn_indices, token_table, bb_table, bb_W, bb_b, rxn_table):
    raise NotImplementedError("write your pallas kernel here")


# speedup vs baseline: 3.0254x; 3.0254x over previous
import functools

import jax
import jax.numpy as jnp
from jax import lax
from jax.experimental import pallas as pl
from jax.experimental.pallas import tpu as pltpu
from jax.experimental.pallas import tpu_sc as plsc

_DIM = 128
_BB_DIM = 64
_B, _L = 4096, 200
_N = _B * _L
_NC, _NS = 2, 16
_NW = _NC * _NS
_CHUNK = 128
_PER_W = _N // _NW
_NCH = _PER_W // _CHUNK
_NBUF = 8
_TS = 1600
_NB = _N // _TS


def _sc_gather(idx2d, table):
  mesh = plsc.VectorSubcoreMesh(
      core_axis_name="c", subcore_axis_name="s",
      num_cores=_NC, num_subcores=_NS)

  @functools.partial(
      pl.kernel,
      out_type=jax.ShapeDtypeStruct((_N, _BB_DIM), jnp.float32),
      mesh=mesh,
      compiler_params=pltpu.CompilerParams(use_tc_tiling_on_sc=False),
      scratch_types=(
          [pltpu.VMEM((_NCH, _CHUNK), jnp.int32),
           pltpu.VMEM((_NBUF, _CHUNK, _BB_DIM), jnp.float32)]
          + [pltpu.SemaphoreType.DMA] * _NBUF
      ),
  )
  def gather_kernel(idx_hbm, table_hbm, out_hbm, idx_v, bufs, *gsems):
    wid = lax.axis_index("s") * _NC + lax.axis_index("c")
    pltpu.sync_copy(idx_hbm.at[pl.ds(wid * _NCH, _NCH)], idx_v)

    def start_gather(c, b):
      pltpu.make_async_copy(
          table_hbm.at[idx_v.at[c]], bufs.at[b], gsems[b]).start()

    def wait_gather(c, b):
      pltpu.make_async_copy(
          table_hbm.at[idx_v.at[c]], bufs.at[b], gsems[b]).wait()

    row0 = wid * _PER_W
    for b in range(_NBUF):
      start_gather(b, b)

    def body(it, carry):
      c0 = it * _NBUF
      for b in range(_NBUF):
        c = c0 + b
        wait_gather(c, b)
        pltpu.sync_copy(bufs.at[b], out_hbm.at[pl.ds(row0 + c * _CHUNK, _CHUNK)])

        @pl.when(c + _NBUF < _NCH)
        def _():
          start_gather(c + _NBUF, b)
      return carry

    lax.fori_loop(0, _NCH // _NBUF, body, 0)

  return gather_kernel(idx2d, table)


def _tc_body(tt_ref, rx_ref, rows_ref, comb_ref, w_ref, b_ref, pe_ref,
             h_ref, m_ref):
  tt = tt_ref[0]
  rx = rx_ref[0]
  rows = rows_ref[0]
  is_bb = tt == 1
  is_rxn = tt == 2
  comb_idx = jnp.where(is_rxn, rx + 8, jnp.where(is_bb, jnp.int32(999), tt))
  ids = lax.broadcasted_iota(jnp.int32, (_TS, 136), 1)
  onehot = (comb_idx == ids).astype(jnp.float32)
  base = jnp.dot(onehot, comb_ref[...], preferred_element_type=jnp.float32)
  mrows = jnp.where(is_bb, rows, 0.0)
  bb = jnp.dot(mrows, w_ref[...], preferred_element_type=jnp.float32)
  h = base + bb + is_bb.astype(jnp.float32) * b_ref[...] + pe_ref[...]
  h_ref[0] = h
  m_ref[0] = jnp.where(tt != 0, 0.0, -jnp.inf).astype(jnp.float32)


def _pe_table(L, d):
  pos = jnp.arange(L, dtype=jnp.float32)[:, None]
  i = jnp.arange(0, d, 2, dtype=jnp.float32)[None, :]
  angle = pos / jnp.power(10000.0, i / d)
  pe = jnp.zeros((L, d), dtype=jnp.float32)
  pe = pe.at[:, 0::2].set(jnp.sin(angle))
  pe = pe.at[:, 1::2].set(jnp.cos(angle))
  return pe


def kernel(token_types, bb_indices, rxn_indices, token_table, bb_table,
           bb_W, bb_b, rxn_table):
  tt = token_types.reshape(-1).astype(jnp.int32)
  bbi = bb_indices.reshape(-1).astype(jnp.int32)
  rxi = rxn_indices.reshape(-1).astype(jnp.int32)

  rows64 = _sc_gather(bbi.reshape(_NW * _NCH, _CHUNK), bb_table)

  comb = jnp.concatenate(
      [token_table, rxn_table,
       jnp.zeros((136 - token_table.shape[0] - rxn_table.shape[0], _DIM),
                 jnp.float32)], axis=0)
  pe_tile = jnp.tile(_pe_table(_L, _DIM), (_TS // _L, 1))

  grid = (_NB,)
  h, m = pl.pallas_call(
      _tc_body,
      grid=grid,
      in_specs=[
          pl.BlockSpec((1, _TS, 1), lambda i: (i, 0, 0)),
          pl.BlockSpec((1, _TS, 1), lambda i: (i, 0, 0)),
          pl.BlockSpec((1, _TS, _BB_DIM), lambda i: (i, 0, 0)),
          pl.BlockSpec((136, _DIM), lambda i: (0, 0)),
          pl.BlockSpec((_BB_DIM, _DIM), lambda i: (0, 0)),
          pl.BlockSpec((1, _DIM), lambda i: (0, 0)),
          pl.BlockSpec((_TS, _DIM), lambda i: (0, 0)),
      ],
      out_specs=[
          pl.BlockSpec((1, _TS, _DIM), lambda i: (i, 0, 0)),
          pl.BlockSpec((1, _TS, 1), lambda i: (i, 0, 0)),
      ],
      out_shape=[
          jax.ShapeDtypeStruct((_NB, _TS, _DIM), jnp.float32),
          jax.ShapeDtypeStruct((_NB, _TS, 1), jnp.float32),
      ],
  )(
      tt.reshape(_NB, _TS, 1),
      rxi.reshape(_NB, _TS, 1),
      rows64.reshape(_NB, _TS, _BB_DIM),
      comb,
      bb_W,
      bb_b.reshape(1, _DIM),
      pe_tile,
  )
  return h.reshape(_B, _L, _DIM), m.reshape(_B, _L)


# speedup vs baseline: 4.8669x; 1.6087x over previous
import functools

import jax
import jax.numpy as jnp
from jax import lax
from jax.experimental import pallas as pl
from jax.experimental.pallas import tpu as pltpu
from jax.experimental.pallas import tpu_sc as plsc

_DIM = 128
_BB_DIM = 64
_B, _L = 4096, 200
_N = _B * _L
_NC, _NS = 2, 16
_NW = _NC * _NS
_CHUNK = 128
_PER_W = _N // _NW
_NCH = _PER_W // _CHUNK
_NBUF = 8
_ROWS_PER_BLK = 8
_TS = _ROWS_PER_BLK * _L
_NB = _B // _ROWS_PER_BLK


_NSTG = _N // 2
_PER_W_STG = _NSTG // _NW
_NCHS = _PER_W_STG // _CHUNK
_NBUF_SC = 4


def _sc_gather(idx_left, idx_right, table):
  mesh = plsc.VectorSubcoreMesh(
      core_axis_name="c", subcore_axis_name="s",
      num_cores=_NC, num_subcores=_NS)

  @functools.partial(
      pl.kernel,
      out_type=jax.ShapeDtypeStruct((_NSTG, _DIM), jnp.float32),
      mesh=mesh,
      compiler_params=pltpu.CompilerParams(use_tc_tiling_on_sc=False),
      scratch_types=(
          [pltpu.VMEM((_NCHS, _CHUNK), jnp.int32),
           pltpu.VMEM((_NCHS, _CHUNK), jnp.int32),
           pltpu.VMEM((_NBUF_SC, _CHUNK, _BB_DIM), jnp.float32),
           pltpu.VMEM((_NBUF_SC, _CHUNK, _BB_DIM), jnp.float32)]
          + [pltpu.SemaphoreType.DMA] * _NBUF_SC
      ),
  )
  def gather_kernel(idxl_hbm, idxr_hbm, table_hbm, out_hbm,
                    idxl_v, idxr_v, bufl, bufr, *gsems):
    wid = lax.axis_index("s") * _NC + lax.axis_index("c")
    pltpu.sync_copy(idxl_hbm.at[pl.ds(wid * _NCHS, _NCHS)], idxl_v)
    pltpu.sync_copy(idxr_hbm.at[pl.ds(wid * _NCHS, _NCHS)], idxr_v)

    def start_gather(c, b):
      pltpu.make_async_copy(
          table_hbm.at[idxl_v.at[c]], bufl.at[b], gsems[b]).start()
      pltpu.make_async_copy(
          table_hbm.at[idxr_v.at[c]], bufr.at[b], gsems[b]).start()

    def wait_gather(c, b):
      pltpu.make_async_copy(
          table_hbm.at[idxl_v.at[c]], bufl.at[b], gsems[b]).wait()
      pltpu.make_async_copy(
          table_hbm.at[idxr_v.at[c]], bufr.at[b], gsems[b]).wait()

    stg0 = wid * _PER_W_STG
    for b in range(_NBUF_SC):
      start_gather(b, b)

    def body(it, carry):
      c0 = it * _NBUF_SC
      for b in range(_NBUF_SC):
        c = c0 + b
        wait_gather(c, b)
        j0 = stg0 + c * _CHUNK
        pltpu.sync_copy(
            bufl.at[b], out_hbm.at[pl.ds(j0, _CHUNK), pl.ds(0, _BB_DIM)])
        pltpu.sync_copy(
            bufr.at[b],
            out_hbm.at[pl.ds(j0, _CHUNK), pl.ds(_BB_DIM, _BB_DIM)])

        @pl.when(c + _NBUF_SC < _NCHS)
        def _():
          start_gather(c + _NBUF_SC, b)
      return carry

    lax.fori_loop(0, _NCHS // _NBUF_SC, body, 0)

  return gather_kernel(idx_left, idx_right, table)


def _tc_body(tt_ref, rx_ref, stg_ref, comb_ref, w_ref, b_ref, pe_ref,
             h_ref, m_ref):
  tt_nat = tt_ref[...]
  rx_nat = rx_ref[...]
  m_ref[...] = jnp.where(tt_nat != 0, 0.0, -jnp.inf).astype(jnp.float32)
  ttT = jnp.transpose(tt_nat)
  rxT = jnp.transpose(rx_nat)

  tt = jnp.concatenate([ttT[:, l:l + 1] for l in range(_ROWS_PER_BLK)], 0)
  rx = jnp.concatenate([rxT[:, l:l + 1] for l in range(_ROWS_PER_BLK)], 0)
  is_bb = tt == 1
  is_rxn = tt == 2
  cidx = jnp.where(is_rxn, rx + 8, jnp.where(is_bb, jnp.int32(999), tt))
  ids = lax.broadcasted_iota(jnp.int32, (_TS, 136), 1)
  onehot = (cidx == ids).astype(jnp.float32)
  base = jnp.dot(onehot, comb_ref[...], preferred_element_type=jnp.float32)

  stg = stg_ref[...]
  halves = []
  for l in range(_ROWS_PER_BLK):
    sl = stg[l * (_L // 2):(l + 1) * (_L // 2), :]
    halves.append(sl[:, :_BB_DIM])
    halves.append(sl[:, _BB_DIM:])
  mrows = jnp.concatenate(halves, 0)
  mrows = jnp.where(is_bb, mrows, 0.0)
  bb = jnp.dot(mrows, w_ref[...], preferred_element_type=jnp.float32)

  h = base + bb + is_bb.astype(jnp.float32) * b_ref[...] + pe_ref[...]
  for l in range(_ROWS_PER_BLK):
    h_ref[l] = h[l * _L:(l + 1) * _L, :]


def _pe_table(L, d):
  pos = jnp.arange(L, dtype=jnp.float32)[:, None]
  i = jnp.arange(0, d, 2, dtype=jnp.float32)[None, :]
  angle = pos / jnp.power(10000.0, i / d)
  pe = jnp.zeros((L, d), dtype=jnp.float32)
  pe = pe.at[:, 0::2].set(jnp.sin(angle))
  pe = pe.at[:, 1::2].set(jnp.cos(angle))
  return pe


def kernel(token_types, bb_indices, rxn_indices, token_table, bb_table,
           bb_W, bb_b, rxn_table):
  tt32 = token_types.astype(jnp.int32)
  rx32 = rxn_indices.astype(jnp.int32)
  bbi = bb_indices.astype(jnp.int32).reshape(_B, 2, _L // 2)
  idx_left = bbi[:, 0, :].reshape(_NW * _NCHS, _CHUNK)
  idx_right = bbi[:, 1, :].reshape(_NW * _NCHS, _CHUNK)

  staging = _sc_gather(idx_left, idx_right, bb_table)

  comb = jnp.concatenate(
      [token_table, rxn_table,
       jnp.zeros((136 - token_table.shape[0] - rxn_table.shape[0], _DIM),
                 jnp.float32)], axis=0)
  pe_tile = jnp.tile(_pe_table(_L, _DIM), (_ROWS_PER_BLK, 1))

  h, m = pl.pallas_call(
      _tc_body,
      grid=(_NB,),
      in_specs=[
          pl.BlockSpec((_ROWS_PER_BLK, _L), lambda i: (i, 0)),
          pl.BlockSpec((_ROWS_PER_BLK, _L), lambda i: (i, 0)),
          pl.BlockSpec((_TS // 2, _DIM), lambda i: (i, 0)),
          pl.BlockSpec((136, _DIM), lambda i: (0, 0)),
          pl.BlockSpec((_BB_DIM, _DIM), lambda i: (0, 0)),
          pl.BlockSpec((1, _DIM), lambda i: (0, 0)),
          pl.BlockSpec((_TS, _DIM), lambda i: (0, 0)),
      ],
      out_specs=[
          pl.BlockSpec((_ROWS_PER_BLK, _L, _DIM), lambda i: (i, 0, 0)),
          pl.BlockSpec((_ROWS_PER_BLK, _L), lambda i: (i, 0)),
      ],
      out_shape=[
          jax.ShapeDtypeStruct((_B, _L, _DIM), jnp.float32),
          jax.ShapeDtypeStruct((_B, _L), jnp.float32),
      ],
  )(tt32, rx32, staging, comb, bb_W, bb_b.reshape(1, _DIM), pe_tile)
  return h, m


# speedup vs baseline: 5.5014x; 1.1304x over previous
import functools

import jax
import jax.numpy as jnp
from jax import lax
from jax.experimental import pallas as pl
from jax.experimental.pallas import tpu as pltpu
from jax.experimental.pallas import tpu_sc as plsc

_DIM = 128
_BB_DIM = 64
_B, _L = 4096, 200
_N = _B * _L
_NC, _NS = 2, 16
_NW = _NC * _NS
_CHUNK = 128
_NSTG = _N // 2
_PER_W_STG = _NSTG // _NW
_NCHS = _PER_W_STG // _CHUNK
_NBUF_SC = 4
_RPB = 16
_TS = _RPB * _L
_NB = _B // _RPB
_BB_SENTINEL = 999


def _sc_gather(idx_left, idx_right, table):
  mesh = plsc.VectorSubcoreMesh(
      core_axis_name="c", subcore_axis_name="s",
      num_cores=_NC, num_subcores=_NS)

  @functools.partial(
      pl.kernel,
      out_type=jax.ShapeDtypeStruct((_NSTG, _DIM), jnp.float32),
      mesh=mesh,
      compiler_params=pltpu.CompilerParams(use_tc_tiling_on_sc=False),
      scratch_types=(
          [pltpu.VMEM((_NCHS, _CHUNK), jnp.int32),
           pltpu.VMEM((_NCHS, _CHUNK), jnp.int32),
           pltpu.VMEM((_NBUF_SC, _CHUNK, _BB_DIM), jnp.float32),
           pltpu.VMEM((_NBUF_SC, _CHUNK, _BB_DIM), jnp.float32)]
          + [pltpu.SemaphoreType.DMA] * _NBUF_SC
      ),
  )
  def gather_kernel(idxl_hbm, idxr_hbm, table_hbm, out_hbm,
                    idxl_v, idxr_v, bufl, bufr, *gsems):
    wid = lax.axis_index("s") * _NC + lax.axis_index("c")
    pltpu.sync_copy(idxl_hbm.at[pl.ds(wid * _NCHS, _NCHS)], idxl_v)
    pltpu.sync_copy(idxr_hbm.at[pl.ds(wid * _NCHS, _NCHS)], idxr_v)

    def start_gather(c, b):
      pltpu.make_async_copy(
          table_hbm.at[idxl_v.at[c]], bufl.at[b], gsems[b]).start()
      pltpu.make_async_copy(
          table_hbm.at[idxr_v.at[c]], bufr.at[b], gsems[b]).start()

    def wait_gather(c, b):
      pltpu.make_async_copy(
          table_hbm.at[idxl_v.at[c]], bufl.at[b], gsems[b]).wait()
      pltpu.make_async_copy(
          table_hbm.at[idxr_v.at[c]], bufr.at[b], gsems[b]).wait()

    stg0 = wid * _PER_W_STG
    for b in range(_NBUF_SC):
      start_gather(b, b)

    def body(it, carry):
      c0 = it * _NBUF_SC
      for b in range(_NBUF_SC):
        c = c0 + b
        wait_gather(c, b)
        j0 = stg0 + c * _CHUNK
        pltpu.sync_copy(
            bufl.at[b], out_hbm.at[pl.ds(j0, _CHUNK), pl.ds(0, _BB_DIM)])
        pltpu.sync_copy(
            bufr.at[b],
            out_hbm.at[pl.ds(j0, _CHUNK), pl.ds(_BB_DIM, _BB_DIM)])

        @pl.when(c + _NBUF_SC < _NCHS)
        def _():
          start_gather(c + _NBUF_SC, b)
      return carry

    lax.fori_loop(0, _NCHS // _NBUF_SC, body, 0)

  return gather_kernel(idx_left, idx_right, table)


def _tc_body(cid_ref, stg_ref, comb_ref, w_ref, b_ref, pe_ref,
             h_ref, m_ref):
  cid_nat = cid_ref[...]
  m_ref[...] = jnp.where(cid_nat != 0, 0.0, -jnp.inf).astype(jnp.float32)

  cidT = jnp.transpose(cid_nat)
  cid = jnp.concatenate([cidT[:, l:l + 1] for l in range(_RPB)], 0)
  is_bb = cid == _BB_SENTINEL
  ids = lax.broadcasted_iota(jnp.int32, (_TS, 136), 1)
  onehot = (cid == ids).astype(jnp.bfloat16)
  base = jnp.dot(onehot, comb_ref[...], preferred_element_type=jnp.float32)

  stg = stg_ref[...]
  halves = []
  for l in range(_RPB):
    sl = stg[l * (_L // 2):(l + 1) * (_L // 2), :]
    halves.append(sl[:, :_BB_DIM])
    halves.append(sl[:, _BB_DIM:])
  mrows = jnp.concatenate(halves, 0)
  mrows = jnp.where(is_bb, mrows, 0.0)
  bb = jnp.dot(mrows, w_ref[...], preferred_element_type=jnp.float32)

  h = base + bb + is_bb.astype(jnp.float32) * b_ref[...] + pe_ref[...]
  for l in range(_RPB):
    h_ref[l] = h[l * _L:(l + 1) * _L, :]


def _pe_table(L, d):
  pos = jnp.arange(L, dtype=jnp.float32)[:, None]
  i = jnp.arange(0, d, 2, dtype=jnp.float32)[None, :]
  angle = pos / jnp.power(10000.0, i / d)
  pe = jnp.zeros((L, d), dtype=jnp.float32)
  pe = pe.at[:, 0::2].set(jnp.sin(angle))
  pe = pe.at[:, 1::2].set(jnp.cos(angle))
  return pe


def kernel(token_types, bb_indices, rxn_indices, token_table, bb_table,
           bb_W, bb_b, rxn_table):
  tt32 = token_types.astype(jnp.int32)
  rx32 = rxn_indices.astype(jnp.int32)
  cidx = jnp.where(tt32 == 2, rx32 + 8,
                   jnp.where(tt32 == 1, _BB_SENTINEL, tt32))

  bbi = bb_indices.astype(jnp.int32).reshape(_B, 2, _L // 2)
  idx_left = bbi[:, 0, :].reshape(_NW * _NCHS, _CHUNK)
  idx_right = bbi[:, 1, :].reshape(_NW * _NCHS, _CHUNK)

  staging = _sc_gather(idx_left, idx_right, bb_table)

  comb = jnp.concatenate(
      [token_table, rxn_table,
       jnp.zeros((136 - token_table.shape[0] - rxn_table.shape[0], _DIM),
                 jnp.float32)], axis=0).astype(jnp.bfloat16)
  pe_tile = jnp.tile(_pe_table(_L, _DIM), (_RPB, 1))

  h, m = pl.pallas_call(
      _tc_body,
      grid=(_NB,),
      in_specs=[
          pl.BlockSpec((_RPB, _L), lambda i: (i, 0)),
          pl.BlockSpec((_TS // 2, _DIM), lambda i: (i, 0)),
          pl.BlockSpec((136, _DIM), lambda i: (0, 0)),
          pl.BlockSpec((_BB_DIM, _DIM), lambda i: (0, 0)),
          pl.BlockSpec((1, _DIM), lambda i: (0, 0)),
          pl.BlockSpec((_TS, _DIM), lambda i: (0, 0)),
      ],
      out_specs=[
          pl.BlockSpec((_RPB, _L, _DIM), lambda i: (i, 0, 0)),
          pl.BlockSpec((_RPB, _L), lambda i: (i, 0)),
      ],
      out_shape=[
          jax.ShapeDtypeStruct((_B, _L, _DIM), jnp.float32),
          jax.ShapeDtypeStruct((_B, _L), jnp.float32),
      ],
  )(cidx, staging, comb, bb_W, bb_b.reshape(1, _DIM), pe_tile)
  return h, m
